# baseline (speedup 1.0000x reference)
import functools

import jax
import jax.numpy as jnp
from jax import lax
from jax.experimental import pallas as pl
from jax.experimental.pallas import tpu as pltpu
from jax.experimental.pallas import tpu_sc as plsc

B, N, K = 2, 10000, 16
CIN, COUT = 64, 64
NB = 10
BN = N // NB

NC, NS = 2, 16
NW = NC * NS
PW0 = 416
PW1 = 864
NPAD = NS * (PW0 + PW1)
CP = 16
CH = (PW0 + PW1) // (2 * CP)
CE = CP * K
GE = 128
NG = CE // GE


def _tc_pre_body(fea_ref, sx_ref, sy_ref, sz_ref, rel_ref, knn_ref, w1_ref,
                 g_ref, idx_ref, cell_ref, rx_ref, ry_ref, rz_ref):
    b = pl.program_id(0)
    w1f = w1_ref[:, :CIN]

    g_ref[...] = lax.dot_general(
        fea_ref[0, 0], w1f, (((0,), (1,)), ((), ())),
        precision=lax.Precision.HIGHEST, preferred_element_type=jnp.float32)

    idx_ref[...] = knn_ref[0, 0] + b * N

    def gidx(v):
        return jnp.clip(jnp.round(((v + 1.0) * 3.0 - 1.0) * 0.5), 0.0, 2.0)
    ixf = gidx(sx_ref[0, 0])
    iyf = gidx(sy_ref[0, 0])
    izf = gidx(sz_ref[0, 0])
    cellf = (izf * 3.0 + iyf) * 3.0 + ixf
    cell_ref[...] = cellf.T.astype(jnp.int32)

    rel = rel_ref[0, 0]
    rx_ref[...] = rel[0].T
    ry_ref[...] = rel[1].T
    rz_ref[...] = rel[2].T


def _tc_pre(fea4, sx4, sy4, sz4, rel5, knn4, w1):
    bkn = pl.BlockSpec((1, 1, K, BN), lambda b, i: (b, i, 0, 0))
    ef = jax.ShapeDtypeStruct((B * N, K), jnp.int32)
    eff = jax.ShapeDtypeStruct((B * N, K), jnp.float32)
    espec = pl.BlockSpec((BN, K), lambda b, i: (b * NB + i, 0))
    return pl.pallas_call(
        _tc_pre_body,
        grid=(B, NB),
        in_specs=[
            pl.BlockSpec((1, 1, CIN, BN), lambda b, i: (b, i, 0, 0)),
            bkn, bkn, bkn,
            pl.BlockSpec((1, 1, 3, K, BN), lambda b, i: (b, i, 0, 0, 0)),
            pl.BlockSpec((1, 1, BN, K), lambda b, i: (b, i, 0, 0)),
            pl.BlockSpec((COUT, CIN + 3), lambda b, i: (0, 0)),
        ],
        out_specs=[
            pl.BlockSpec((BN, COUT), lambda b, i: (b * NB + i, 0)),
            espec, espec, espec, espec, espec,
        ],
        out_shape=[
            jax.ShapeDtypeStruct((B * N, COUT), jnp.float32),
            ef, ef, eff, eff, eff,
        ],
    )(fea4, sx4, sy4, sz4, rel5, knn4, w1)


_BCAST_DN = lax.GatherDimensionNumbers(
    offset_dims=(), collapsed_slice_dims=(0,), start_index_map=(0,))


def _lane_bcast(vec, k):
    idx = jnp.full((16, 1), k, jnp.int32)
    return lax.gather(vec, idx, _BCAST_DN, slice_sizes=(1,),
                      mode=lax.GatherScatterMode.PROMISE_IN_BOUNDS)


def _sc_body(g_hbm, idx_hbm, cell_hbm, rx_hbm, ry_hbm, rz_hbm, wtab_hbm,
             w1xt_hbm, out_hbm,
             idx_v0, idx_v1, cell_v0, cell_v1, rx_v0, rx_v1, ry_v0, ry_v1,
             rz_v0, rz_v1, rows_v0, rows_v1, out_v0, out_v1, wtab_v, w1xt_v,
             si0, si1, sg0, sg1, so0, so1):
    idx_v = (idx_v0, idx_v1)
    cell_v = (cell_v0, cell_v1)
    rx_v = (rx_v0, rx_v1)
    ry_v = (ry_v0, ry_v1)
    rz_v = (rz_v0, rz_v1)
    rows_v = (rows_v0, rows_v1)
    out_v = (out_v0, out_v1)
    si = (si0, si1)
    sg = (sg0, sg1)
    so = (so0, so1)

    cid = lax.axis_index("c")
    sid = lax.axis_index("s")
    pb0 = jnp.where(cid == 0, sid * PW0, NS * PW0 + sid * PW1)
    nch = jnp.where(cid == 0, PW0 // CP, PW1 // CP)
    pltpu.sync_copy(wtab_hbm, wtab_v)
    pltpu.sync_copy(w1xt_hbm, w1xt_v)
    w1r = [[w1xt_v[d, pl.ds(j * 16, 16)] for j in range(4)] for d in range(3)]

    def issue_in(c, b):
        pbase = pb0 + c * CP
        pltpu.async_copy(idx_hbm.at[pl.ds(pbase * K, CE)], idx_v[b], si[b])
        pltpu.async_copy(cell_hbm.at[pl.ds(pbase, CP)], cell_v[b], si[b])
        pltpu.async_copy(rx_hbm.at[pl.ds(pbase, CP)], rx_v[b], si[b])
        pltpu.async_copy(ry_hbm.at[pl.ds(pbase, CP)], ry_v[b], si[b])
        pltpu.async_copy(rz_hbm.at[pl.ds(pbase, CP)], rz_v[b], si[b])

    def wait_in(b):
        pltpu.make_async_copy(idx_hbm.at[pl.ds(0, CE)], idx_v[b], si[b]).wait()
        pltpu.make_async_copy(cell_hbm.at[pl.ds(0, CP)], cell_v[b], si[b]).wait()
        pltpu.make_async_copy(rx_hbm.at[pl.ds(0, CP)], rx_v[b], si[b]).wait()
        pltpu.make_async_copy(ry_hbm.at[pl.ds(0, CP)], ry_v[b], si[b]).wait()
        pltpu.make_async_copy(rz_hbm.at[pl.ds(0, CP)], rz_v[b], si[b]).wait()

    def issue_gather(b):
        for h in range(NG):
            hs = pl.ds(h * GE, GE)
            pltpu.async_copy(g_hbm.at[idx_v[b].at[hs]], rows_v[b].at[hs], sg[b])

    def wait_gather(b):
        for h in range(NG):
            hs = pl.ds(h * GE, GE)
            pltpu.make_async_copy(g_hbm.at[idx_v[b].at[hs]],
                                  rows_v[b].at[hs], sg[b]).wait()

    def issue_out(c, b):
        pbase = pb0 + c * CP
        pltpu.async_copy(out_v[b], out_hbm.at[pl.ds(pbase, CP)], so[b])

    def wait_out(b):
        pltpu.make_async_copy(out_v[b], out_hbm.at[pl.ds(0, CP)], so[b]).wait()

    def compute(b):
        def point_body(p, pcarry):
            base = p * K
            cv = cell_v[b][p]
            rxv = rx_v[b][p]
            ryv = ry_v[b][p]
            rzv = rz_v[b][p]
            accs = [jnp.zeros((16,), jnp.float32) for _ in range(4)]
            for k in range(K):
                cl = cv[k]
                rxb = _lane_bcast(rxv, k)
                ryb = _lane_bcast(ryv, k)
                rzb = _lane_bcast(rzv, k)
                row = base + k
                for j in range(4):
                    jds = pl.ds(j * 16, 16)
                    w = wtab_v[cl, jds]
                    u = (rows_v[b][row, jds] + rxb * w1r[0][j] +
                         ryb * w1r[1][j] + rzb * w1r[2][j])
                    accs[j] = accs[j] + w * u
            for j in range(4):
                out_v[b][p, pl.ds(j * 16, 16)] = accs[j]
            return pcarry

        lax.fori_loop(0, CP, point_body, 0)

    issue_in(0, 0)
    wait_in(0)
    issue_gather(0)
    issue_in(1, 1)

    def body2(c2, carry):
        for b in range(2):
            c = c2 * 2 + b
            nb = 1 - b
            wait_gather(b)

            @pl.when(c + 1 < nch)
            def _():
                wait_in(nb)
                issue_gather(nb)

            @pl.when(c >= 2)
            def _():
                wait_out(b)

            compute(b)
            issue_out(c, b)

            @pl.when(c + 2 < nch)
            def _():
                issue_in(c + 2, b)
        return carry

    lax.fori_loop(0, nch // 2, body2, 0)
    wait_out(0)
    wait_out(1)


def _sc_gather_combine(g, idx_flat, cell_pad, rx, ry, rz, wtab, w1xt):
    mesh = plsc.VectorSubcoreMesh(core_axis_name="c", subcore_axis_name="s")
    cpk = pltpu.VMEM((CP, K), jnp.int32)
    cpf = pltpu.VMEM((CP, K), jnp.float32)
    f = functools.partial(
        pl.kernel,
        mesh=mesh,
        compiler_params=pltpu.CompilerParams(use_tc_tiling_on_sc=False),
        out_type=jax.ShapeDtypeStruct((NPAD, COUT), jnp.float32),
        scratch_types=[
            pltpu.VMEM((CE,), jnp.int32), pltpu.VMEM((CE,), jnp.int32),
            cpk, cpk,
            cpf, cpf, cpf, cpf, cpf, cpf,
            pltpu.VMEM((CE, COUT), jnp.float32),
            pltpu.VMEM((CE, COUT), jnp.float32),
            pltpu.VMEM((CP, COUT), jnp.float32),
            pltpu.VMEM((CP, COUT), jnp.float32),
            pltpu.VMEM((27, COUT), jnp.float32),
            pltpu.VMEM((3, COUT), jnp.float32),
            pltpu.SemaphoreType.DMA, pltpu.SemaphoreType.DMA,
            pltpu.SemaphoreType.DMA, pltpu.SemaphoreType.DMA,
            pltpu.SemaphoreType.DMA, pltpu.SemaphoreType.DMA,
        ],
    )(_sc_body)
    return f(g, idx_flat, cell_pad, rx, ry, rz, wtab, w1xt)


def _tc_post_body(sc_ref, out_ref):
    out_ref[0] = sc_ref[...].T


def _tc_post(out_sc):
    return pl.pallas_call(
        _tc_post_body,
        grid=(B,),
        in_specs=[pl.BlockSpec((N, COUT), lambda b: (b, 0))],
        out_specs=pl.BlockSpec((1, COUT, N), lambda b: (b, 0, 0)),
        out_shape=jax.ShapeDtypeStruct((B, COUT, N), jnp.float32),
    )(out_sc)


def kernel(rel_xyz, sample_xyz, fea, knn_idx, conv_dw, W1):
    wtab = conv_dw.reshape(COUT, 27).T
    w1xt = W1[:, CIN:].T
    sq = jnp.squeeze(sample_xyz, 3)
    sx4 = jnp.transpose(sq[..., 0].reshape(B, K, NB, BN), (0, 2, 1, 3))
    sy4 = jnp.transpose(sq[..., 1].reshape(B, K, NB, BN), (0, 2, 1, 3))
    sz4 = jnp.transpose(sq[..., 2].reshape(B, K, NB, BN), (0, 2, 1, 3))
    rel5 = jnp.transpose(rel_xyz.reshape(B, 3, K, NB, BN), (0, 3, 1, 2, 4))
    fea4 = jnp.transpose(fea.reshape(B, CIN, NB, BN), (0, 2, 1, 3))
    knn4 = knn_idx.reshape(B, NB, BN, K)

    g, idx_e, cell_e, rx_e, ry_e, rz_e = _tc_pre(
        fea4, sx4, sy4, sz4, rel5, knn4, W1)

    pad = ((0, NPAD - B * N), (0, 0))
    idx_flat = jnp.pad(idx_e, pad).reshape(NPAD * K)
    cell_pad = jnp.pad(cell_e, pad)
    rx = jnp.pad(rx_e, pad)
    ry = jnp.pad(ry_e, pad)
    rz = jnp.pad(rz_e, pad)

    out_sc = _sc_gather_combine(
        g, idx_flat, cell_pad, rx, ry, rz, wtab, w1xt)[:B * N]
    return _tc_post(out_sc)


# speedup vs baseline: 1.1568x; 1.1568x over previous
import functools

import jax
import jax.numpy as jnp
from jax import lax
from jax.experimental import pallas as pl
from jax.experimental.pallas import tpu as pltpu
from jax.experimental.pallas import tpu_sc as plsc

B, N, K = 2, 10000, 16
CIN, COUT = 64, 64
NB = 10
BN = N // NB

NC, NS = 2, 16
NW = NC * NS
PW0 = 864
PW1 = 416
NPAD = NS * (PW0 + PW1)
CP = 16
CH = (PW0 + PW1) // (2 * CP)
CE = CP * K
GE = 128
NG = CE // GE


def _tc_pre_body(fea_ref, sx_ref, sy_ref, sz_ref, rel_ref, knn_ref, w1_ref,
                 g_ref, idx_ref, cell_ref, rx_ref, ry_ref, rz_ref):
    b = pl.program_id(0)
    w1f = w1_ref[:, :CIN]

    g_ref[...] = lax.dot_general(
        fea_ref[0, 0], w1f, (((0,), (1,)), ((), ())),
        precision=lax.Precision.HIGHEST, preferred_element_type=jnp.float32)

    idx_ref[...] = knn_ref[0, 0] + b * N

    def gidx(v):
        return jnp.clip(jnp.round(((v + 1.0) * 3.0 - 1.0) * 0.5), 0.0, 2.0)
    ixf = gidx(sx_ref[0, 0])
    iyf = gidx(sy_ref[0, 0])
    izf = gidx(sz_ref[0, 0])
    cellf = (izf * 3.0 + iyf) * 3.0 + ixf
    cell_ref[...] = cellf.T.astype(jnp.int32)

    rel = rel_ref[0, 0]
    rx_ref[...] = rel[0].T
    ry_ref[...] = rel[1].T
    rz_ref[...] = rel[2].T


def _tc_pre(fea4, sx4, sy4, sz4, rel5, knn4, w1):
    bkn = pl.BlockSpec((1, 1, K, BN), lambda b, i: (b, i, 0, 0))
    ef = jax.ShapeDtypeStruct((B * N, K), jnp.int32)
    eff = jax.ShapeDtypeStruct((B * N, K), jnp.float32)
    espec = pl.BlockSpec((BN, K), lambda b, i: (b * NB + i, 0))
    return pl.pallas_call(
        _tc_pre_body,
        grid=(B, NB),
        in_specs=[
            pl.BlockSpec((1, 1, CIN, BN), lambda b, i: (b, i, 0, 0)),
            bkn, bkn, bkn,
            pl.BlockSpec((1, 1, 3, K, BN), lambda b, i: (b, i, 0, 0, 0)),
            pl.BlockSpec((1, 1, BN, K), lambda b, i: (b, i, 0, 0)),
            pl.BlockSpec((COUT, CIN + 3), lambda b, i: (0, 0)),
        ],
        out_specs=[
            pl.BlockSpec((BN, COUT), lambda b, i: (b * NB + i, 0)),
            espec, espec, espec, espec, espec,
        ],
        out_shape=[
            jax.ShapeDtypeStruct((B * N, COUT), jnp.float32),
            ef, ef, eff, eff, eff,
        ],
    )(fea4, sx4, sy4, sz4, rel5, knn4, w1)


_BCAST_DN = lax.GatherDimensionNumbers(
    offset_dims=(), collapsed_slice_dims=(0,), start_index_map=(0,))


def _lane_bcast(vec, k):
    idx = jnp.full((16, 1), k, jnp.int32)
    return lax.gather(vec, idx, _BCAST_DN, slice_sizes=(1,),
                      mode=lax.GatherScatterMode.PROMISE_IN_BOUNDS)


def _sc_body(g_hbm, idx_hbm, cell_hbm, rx_hbm, ry_hbm, rz_hbm, wtab_hbm,
             w1xt_hbm, out_hbm,
             idx_v0, idx_v1, cell_v0, cell_v1, rx_v0, rx_v1, ry_v0, ry_v1,
             rz_v0, rz_v1, rows_v0, rows_v1, out_v0, out_v1, wtab_v, w1xt_v,
             si0, si1, sg0, sg1, so0, so1):
    idx_v = (idx_v0, idx_v1)
    cell_v = (cell_v0, cell_v1)
    rx_v = (rx_v0, rx_v1)
    ry_v = (ry_v0, ry_v1)
    rz_v = (rz_v0, rz_v1)
    rows_v = (rows_v0, rows_v1)
    out_v = (out_v0, out_v1)
    si = (si0, si1)
    sg = (sg0, sg1)
    so = (so0, so1)

    cid = lax.axis_index("c")
    sid = lax.axis_index("s")
    pb0 = jnp.where(cid == 0, sid * PW0, NS * PW0 + sid * PW1)
    nch = jnp.where(cid == 0, PW0 // CP, PW1 // CP)
    pltpu.sync_copy(wtab_hbm, wtab_v)
    pltpu.sync_copy(w1xt_hbm, w1xt_v)
    w1r = [[w1xt_v[d, pl.ds(j * 16, 16)] for j in range(4)] for d in range(3)]

    def issue_in(c, b):
        pbase = pb0 + c * CP
        pltpu.async_copy(idx_hbm.at[pl.ds(pbase * K, CE)], idx_v[b], si[b])
        pltpu.async_copy(cell_hbm.at[pl.ds(pbase, CP)], cell_v[b], si[b])
        pltpu.async_copy(rx_hbm.at[pl.ds(pbase, CP)], rx_v[b], si[b])
        pltpu.async_copy(ry_hbm.at[pl.ds(pbase, CP)], ry_v[b], si[b])
        pltpu.async_copy(rz_hbm.at[pl.ds(pbase, CP)], rz_v[b], si[b])

    def wait_in(b):
        pltpu.make_async_copy(idx_hbm.at[pl.ds(0, CE)], idx_v[b], si[b]).wait()
        pltpu.make_async_copy(cell_hbm.at[pl.ds(0, CP)], cell_v[b], si[b]).wait()
        pltpu.make_async_copy(rx_hbm.at[pl.ds(0, CP)], rx_v[b], si[b]).wait()
        pltpu.make_async_copy(ry_hbm.at[pl.ds(0, CP)], ry_v[b], si[b]).wait()
        pltpu.make_async_copy(rz_hbm.at[pl.ds(0, CP)], rz_v[b], si[b]).wait()

    def issue_gather(b):
        for h in range(NG):
            hs = pl.ds(h * GE, GE)
            pltpu.async_copy(g_hbm.at[idx_v[b].at[hs]], rows_v[b].at[hs], sg[b])

    def wait_gather(b):
        for h in range(NG):
            hs = pl.ds(h * GE, GE)
            pltpu.make_async_copy(g_hbm.at[idx_v[b].at[hs]],
                                  rows_v[b].at[hs], sg[b]).wait()

    def issue_out(c, b):
        pbase = pb0 + c * CP
        pltpu.async_copy(out_v[b], out_hbm.at[pl.ds(pbase, CP)], so[b])

    def wait_out(b):
        pltpu.make_async_copy(out_v[b], out_hbm.at[pl.ds(0, CP)], so[b]).wait()

    def compute(b):
        def point_body(p, pcarry):
            base = p * K
            cv = cell_v[b][p]
            rxv = rx_v[b][p]
            ryv = ry_v[b][p]
            rzv = rz_v[b][p]
            accs = [jnp.zeros((16,), jnp.float32) for _ in range(4)]
            for k in range(K):
                cl = cv[k]
                rxb = _lane_bcast(rxv, k)
                ryb = _lane_bcast(ryv, k)
                rzb = _lane_bcast(rzv, k)
                row = base + k
                for j in range(4):
                    jds = pl.ds(j * 16, 16)
                    w = wtab_v[cl, jds]
                    u = (rows_v[b][row, jds] + rxb * w1r[0][j] +
                         ryb * w1r[1][j] + rzb * w1r[2][j])
                    accs[j] = accs[j] + w * u
            for j in range(4):
                out_v[b][p, pl.ds(j * 16, 16)] = accs[j]
            return pcarry

        lax.fori_loop(0, CP, point_body, 0)

    issue_in(0, 0)
    wait_in(0)
    issue_gather(0)
    issue_in(1, 1)

    def body2(c2, carry):
        for b in range(2):
            c = c2 * 2 + b
            nb = 1 - b
            wait_gather(b)

            @pl.when(c + 1 < nch)
            def _():
                wait_in(nb)
                issue_gather(nb)

            @pl.when(c >= 2)
            def _():
                wait_out(b)

            compute(b)
            issue_out(c, b)

            @pl.when(c + 2 < nch)
            def _():
                issue_in(c + 2, b)
        return carry

    lax.fori_loop(0, nch // 2, body2, 0)
    wait_out(0)
    wait_out(1)


def _sc_gather_combine(g, idx_flat, cell_pad, rx, ry, rz, wtab, w1xt):
    mesh = plsc.VectorSubcoreMesh(core_axis_name="c", subcore_axis_name="s")
    cpk = pltpu.VMEM((CP, K), jnp.int32)
    cpf = pltpu.VMEM((CP, K), jnp.float32)
    f = functools.partial(
        pl.kernel,
        mesh=mesh,
        compiler_params=pltpu.CompilerParams(use_tc_tiling_on_sc=False),
        out_type=jax.ShapeDtypeStruct((NPAD, COUT), jnp.float32),
        scratch_types=[
            pltpu.VMEM((CE,), jnp.int32), pltpu.VMEM((CE,), jnp.int32),
            cpk, cpk,
            cpf, cpf, cpf, cpf, cpf, cpf,
            pltpu.VMEM((CE, COUT), jnp.float32),
            pltpu.VMEM((CE, COUT), jnp.float32),
            pltpu.VMEM((CP, COUT), jnp.float32),
            pltpu.VMEM((CP, COUT), jnp.float32),
            pltpu.VMEM((27, COUT), jnp.float32),
            pltpu.VMEM((3, COUT), jnp.float32),
            pltpu.SemaphoreType.DMA, pltpu.SemaphoreType.DMA,
            pltpu.SemaphoreType.DMA, pltpu.SemaphoreType.DMA,
            pltpu.SemaphoreType.DMA, pltpu.SemaphoreType.DMA,
        ],
    )(_sc_body)
    return f(g, idx_flat, cell_pad, rx, ry, rz, wtab, w1xt)


def _tc_post_body(sc_ref, out_ref):
    out_ref[0] = sc_ref[...].T


def _tc_post(out_sc):
    return pl.pallas_call(
        _tc_post_body,
        grid=(B,),
        in_specs=[pl.BlockSpec((N, COUT), lambda b: (b, 0))],
        out_specs=pl.BlockSpec((1, COUT, N), lambda b: (b, 0, 0)),
        out_shape=jax.ShapeDtypeStruct((B, COUT, N), jnp.float32),
    )(out_sc)


def kernel(rel_xyz, sample_xyz, fea, knn_idx, conv_dw, W1):
    wtab = conv_dw.reshape(COUT, 27).T
    w1xt = W1[:, CIN:].T
    sq = jnp.squeeze(sample_xyz, 3)
    sx4 = jnp.transpose(sq[..., 0].reshape(B, K, NB, BN), (0, 2, 1, 3))
    sy4 = jnp.transpose(sq[..., 1].reshape(B, K, NB, BN), (0, 2, 1, 3))
    sz4 = jnp.transpose(sq[..., 2].reshape(B, K, NB, BN), (0, 2, 1, 3))
    rel5 = jnp.transpose(rel_xyz.reshape(B, 3, K, NB, BN), (0, 3, 1, 2, 4))
    fea4 = jnp.transpose(fea.reshape(B, CIN, NB, BN), (0, 2, 1, 3))
    knn4 = knn_idx.reshape(B, NB, BN, K)

    g, idx_e, cell_e, rx_e, ry_e, rz_e = _tc_pre(
        fea4, sx4, sy4, sz4, rel5, knn4, W1)

    pad = ((0, NPAD - B * N), (0, 0))
    idx_flat = jnp.pad(idx_e, pad).reshape(NPAD * K)
    cell_pad = jnp.pad(cell_e, pad)
    rx = jnp.pad(rx_e, pad)
    ry = jnp.pad(ry_e, pad)
    rz = jnp.pad(rz_e, pad)

    out_sc = _sc_gather_combine(
        g, idx_flat, cell_pad, rx, ry, rz, wtab, w1xt)[:B * N]
    return _tc_post(out_sc)


# speedup vs baseline: 1.5789x; 1.3649x over previous
import functools

import jax
import jax.numpy as jnp
from jax import lax
from jax.experimental import pallas as pl
from jax.experimental.pallas import tpu as pltpu
from jax.experimental.pallas import tpu_sc as plsc

B, N, K = 2, 10000, 16
CIN, COUT = 64, 64
NB = 10
BN = N // NB

NC, NS = 2, 16
NW = NC * NS
PW0 = 928
PW1 = 320
PW1L = 352
C1BASE = NS * PW0
CP = 16
CE = CP * K
GE = 128
NG = CE // GE


def _tc_pre_body(fea_ref, sx_ref, sy_ref, sz_ref, rel_ref, knn_ref, w1_ref,
                 g_ref, idx_ref, cell_ref, rx_ref, ry_ref, rz_ref):
    b = pl.program_id(0)
    w1f = w1_ref[:, :CIN]

    g_ref[...] = lax.dot_general(
        fea_ref[0, 0], w1f, (((0,), (1,)), ((), ())),
        precision=lax.Precision.HIGHEST, preferred_element_type=jnp.float32)

    idx_ref[...] = knn_ref[0, 0] + b * N

    def gidx(v):
        return jnp.clip(jnp.round(((v + 1.0) * 3.0 - 1.0) * 0.5), 0.0, 2.0)
    ixf = gidx(sx_ref[0, 0])
    iyf = gidx(sy_ref[0, 0])
    izf = gidx(sz_ref[0, 0])
    cellf = (izf * 3.0 + iyf) * 3.0 + ixf
    cell_ref[...] = cellf.T.astype(jnp.int32)

    rel = rel_ref[0, 0]
    rx_ref[...] = rel[0].T
    ry_ref[...] = rel[1].T
    rz_ref[...] = rel[2].T


def _tc_pre(fea4, sx4, sy4, sz4, rel5, knn4, w1):
    bkn = pl.BlockSpec((1, 1, K, BN), lambda b, i: (b, i, 0, 0))
    ef = jax.ShapeDtypeStruct((B * N, K), jnp.int32)
    eff = jax.ShapeDtypeStruct((B * N, K), jnp.float32)
    espec = pl.BlockSpec((BN, K), lambda b, i: (b * NB + i, 0))
    return pl.pallas_call(
        _tc_pre_body,
        grid=(B, NB),
        in_specs=[
            pl.BlockSpec((1, 1, CIN, BN), lambda b, i: (b, i, 0, 0)),
            bkn, bkn, bkn,
            pl.BlockSpec((1, 1, 3, K, BN), lambda b, i: (b, i, 0, 0, 0)),
            pl.BlockSpec((1, 1, BN, K), lambda b, i: (b, i, 0, 0)),
            pl.BlockSpec((COUT, CIN + 3), lambda b, i: (0, 0)),
        ],
        out_specs=[
            pl.BlockSpec((BN, COUT), lambda b, i: (b * NB + i, 0)),
            espec, espec, espec, espec, espec,
        ],
        out_shape=[
            jax.ShapeDtypeStruct((B * N, COUT), jnp.float32),
            ef, ef, eff, eff, eff,
        ],
    )(fea4, sx4, sy4, sz4, rel5, knn4, w1)


_BCAST_DN = lax.GatherDimensionNumbers(
    offset_dims=(), collapsed_slice_dims=(0,), start_index_map=(0,))


def _lane_bcast(vec, k):
    idx = jnp.full((16, 1), k, jnp.int32)
    return lax.gather(vec, idx, _BCAST_DN, slice_sizes=(1,),
                      mode=lax.GatherScatterMode.PROMISE_IN_BOUNDS)


def _sc_body(g_hbm, idx_hbm, cell_hbm, rx_hbm, ry_hbm, rz_hbm, wtab_hbm,
             w1xt_hbm, out_hbm,
             idx_v0, idx_v1, cell_v0, cell_v1, rx_v0, rx_v1, ry_v0, ry_v1,
             rz_v0, rz_v1, rows_v0, rows_v1, out_v0, out_v1, wtab_v, w1xt_v,
             si0, si1, sg0, sg1, so0, so1):
    idx_v = (idx_v0, idx_v1)
    cell_v = (cell_v0, cell_v1)
    rx_v = (rx_v0, rx_v1)
    ry_v = (ry_v0, ry_v1)
    rz_v = (rz_v0, rz_v1)
    rows_v = (rows_v0, rows_v1)
    out_v = (out_v0, out_v1)
    si = (si0, si1)
    sg = (sg0, sg1)
    so = (so0, so1)

    cid = lax.axis_index("c")
    sid = lax.axis_index("s")
    pb0 = jnp.where(cid == 0, sid * PW0, C1BASE + sid * PW1)
    nch = jnp.where(cid == 0, PW0 // CP,
                    jnp.where(sid == NS - 1, PW1L // CP, PW1 // CP))
    pltpu.sync_copy(wtab_hbm, wtab_v)
    pltpu.sync_copy(w1xt_hbm, w1xt_v)
    w1r = [[w1xt_v[d, pl.ds(j * 16, 16)] for j in range(4)] for d in range(3)]

    def issue_in(c, b):
        pbase = pb0 + c * CP
        pltpu.async_copy(idx_hbm.at[pl.ds(pbase * K, CE)], idx_v[b], si[b])
        pltpu.async_copy(cell_hbm.at[pl.ds(pbase, CP)], cell_v[b], si[b])
        pltpu.async_copy(rx_hbm.at[pl.ds(pbase, CP)], rx_v[b], si[b])
        pltpu.async_copy(ry_hbm.at[pl.ds(pbase, CP)], ry_v[b], si[b])
        pltpu.async_copy(rz_hbm.at[pl.ds(pbase, CP)], rz_v[b], si[b])

    def wait_in(b):
        pltpu.make_async_copy(idx_hbm.at[pl.ds(0, CE)], idx_v[b], si[b]).wait()
        pltpu.make_async_copy(cell_hbm.at[pl.ds(0, CP)], cell_v[b], si[b]).wait()
        pltpu.make_async_copy(rx_hbm.at[pl.ds(0, CP)], rx_v[b], si[b]).wait()
        pltpu.make_async_copy(ry_hbm.at[pl.ds(0, CP)], ry_v[b], si[b]).wait()
        pltpu.make_async_copy(rz_hbm.at[pl.ds(0, CP)], rz_v[b], si[b]).wait()

    def issue_gather(b):
        for h in range(NG):
            hs = pl.ds(h * GE, GE)
            pltpu.async_copy(g_hbm.at[idx_v[b].at[hs]], rows_v[b].at[hs], sg[b])

    def wait_gather(b):
        for h in range(NG):
            hs = pl.ds(h * GE, GE)
            pltpu.make_async_copy(g_hbm.at[idx_v[b].at[hs]],
                                  rows_v[b].at[hs], sg[b]).wait()

    def issue_out(c, b):
        pbase = pb0 + c * CP
        pltpu.async_copy(out_v[b], out_hbm.at[pl.ds(pbase, CP)], so[b])

    def wait_out(b):
        pltpu.make_async_copy(out_v[b], out_hbm.at[pl.ds(0, CP)], so[b]).wait()

    def compute(b):
        def point_body(p, pcarry):
            base = p * K
            cv = cell_v[b][p]
            rxv = rx_v[b][p]
            ryv = ry_v[b][p]
            rzv = rz_v[b][p]
            accs = [jnp.zeros((16,), jnp.float32) for _ in range(4)]
            for k in range(K):
                cl = cv[k]
                rxb = _lane_bcast(rxv, k)
                ryb = _lane_bcast(ryv, k)
                rzb = _lane_bcast(rzv, k)
                row = base + k
                for j in range(4):
                    jds = pl.ds(j * 16, 16)
                    w = wtab_v[cl, jds]
                    u = (rows_v[b][row, jds] + rxb * w1r[0][j] +
                         ryb * w1r[1][j] + rzb * w1r[2][j])
                    accs[j] = accs[j] + w * u
            for j in range(4):
                out_v[b][p, pl.ds(j * 16, 16)] = accs[j]
            return pcarry

        lax.fori_loop(0, CP, point_body, 0)

    issue_in(0, 0)
    wait_in(0)
    issue_gather(0)
    issue_in(1, 1)

    def body2(c2, carry):
        for b in range(2):
            c = c2 * 2 + b
            nb = 1 - b
            wait_gather(b)

            @pl.when(c + 1 < nch)
            def _():
                wait_in(nb)
                issue_gather(nb)

            @pl.when(c >= 2)
            def _():
                wait_out(b)

            compute(b)
            issue_out(c, b)

            @pl.when(c + 2 < nch)
            def _():
                issue_in(c + 2, b)
        return carry

    lax.fori_loop(0, nch // 2, body2, 0)
    wait_out(0)
    wait_out(1)


def _sc_gather_combine(g, idx_flat, cell_pad, rx, ry, rz, wtab, w1xt):
    mesh = plsc.VectorSubcoreMesh(core_axis_name="c", subcore_axis_name="s")
    cpk = pltpu.VMEM((CP, K), jnp.int32)
    cpf = pltpu.VMEM((CP, K), jnp.float32)
    f = functools.partial(
        pl.kernel,
        mesh=mesh,
        compiler_params=pltpu.CompilerParams(use_tc_tiling_on_sc=False),
        out_type=jax.ShapeDtypeStruct((B * N, COUT), jnp.float32),
        scratch_types=[
            pltpu.VMEM((CE,), jnp.int32), pltpu.VMEM((CE,), jnp.int32),
            cpk, cpk,
            cpf, cpf, cpf, cpf, cpf, cpf,
            pltpu.VMEM((CE, COUT), jnp.float32),
            pltpu.VMEM((CE, COUT), jnp.float32),
            pltpu.VMEM((CP, COUT), jnp.float32),
            pltpu.VMEM((CP, COUT), jnp.float32),
            pltpu.VMEM((27, COUT), jnp.float32),
            pltpu.VMEM((3, COUT), jnp.float32),
            pltpu.SemaphoreType.DMA, pltpu.SemaphoreType.DMA,
            pltpu.SemaphoreType.DMA, pltpu.SemaphoreType.DMA,
            pltpu.SemaphoreType.DMA, pltpu.SemaphoreType.DMA,
        ],
    )(_sc_body)
    return f(g, idx_flat, cell_pad, rx, ry, rz, wtab, w1xt)


def _tc_post_body(sc_ref, out_ref):
    out_ref[0] = sc_ref[...].T


def _tc_post(out_sc):
    return pl.pallas_call(
        _tc_post_body,
        grid=(B,),
        in_specs=[pl.BlockSpec((N, COUT), lambda b: (b, 0))],
        out_specs=pl.BlockSpec((1, COUT, N), lambda b: (b, 0, 0)),
        out_shape=jax.ShapeDtypeStruct((B, COUT, N), jnp.float32),
    )(out_sc)


def kernel(rel_xyz, sample_xyz, fea, knn_idx, conv_dw, W1):
    wtab = conv_dw.reshape(COUT, 27).T
    w1xt = W1[:, CIN:].T
    sq = jnp.squeeze(sample_xyz, 3)
    sx4 = jnp.transpose(sq[..., 0].reshape(B, K, NB, BN), (0, 2, 1, 3))
    sy4 = jnp.transpose(sq[..., 1].reshape(B, K, NB, BN), (0, 2, 1, 3))
    sz4 = jnp.transpose(sq[..., 2].reshape(B, K, NB, BN), (0, 2, 1, 3))
    rel5 = jnp.transpose(rel_xyz.reshape(B, 3, K, NB, BN), (0, 3, 1, 2, 4))
    fea4 = jnp.transpose(fea.reshape(B, CIN, NB, BN), (0, 2, 1, 3))
    knn4 = knn_idx.reshape(B, NB, BN, K)

    g, idx_e, cell_e, rx_e, ry_e, rz_e = _tc_pre(
        fea4, sx4, sy4, sz4, rel5, knn4, W1)

    idx_flat = idx_e.reshape(B * N * K)
    out_sc = _sc_gather_combine(
        g, idx_flat, cell_e, rx_e, ry_e, rz_e, wtab, w1xt)
    return _tc_post(out_sc)


# speedup vs baseline: 1.8395x; 1.1650x over previous
import functools

import jax
import jax.numpy as jnp
from jax import lax
from jax.experimental import pallas as pl
from jax.experimental.pallas import tpu as pltpu
from jax.experimental.pallas import tpu_sc as plsc

B, N, K = 2, 10000, 16
CIN, COUT = 64, 64
NB = 10
BN = N // NB

NC, NS = 2, 16
NW = NC * NS
CP = 16
CE = CP * K
GE = 128
NG = CE // GE


def _tc_pre_body(fea_ref, sx_ref, sy_ref, sz_ref, rel_ref, knn_ref, w1_ref,
                 g_ref, idx_ref, cell_ref, rx_ref, ry_ref, rz_ref):
    b = pl.program_id(0)
    w1f = w1_ref[:, :CIN]

    g_ref[...] = lax.dot_general(
        fea_ref[0, 0], w1f, (((0,), (1,)), ((), ())),
        precision=lax.Precision.HIGHEST, preferred_element_type=jnp.float32)

    idx_ref[...] = knn_ref[0, 0] + b * N

    def gidx(v):
        return jnp.clip(jnp.round(((v + 1.0) * 3.0 - 1.0) * 0.5), 0.0, 2.0)
    ixf = gidx(sx_ref[0, 0])
    iyf = gidx(sy_ref[0, 0])
    izf = gidx(sz_ref[0, 0])
    cellf = (izf * 3.0 + iyf) * 3.0 + ixf
    cell_ref[...] = cellf.T.astype(jnp.int32)

    rel = rel_ref[0, 0]
    rx_ref[...] = rel[0].T
    ry_ref[...] = rel[1].T
    rz_ref[...] = rel[2].T


def _tc_pre(fea4, sx4, sy4, sz4, rel5, knn4, w1):
    bkn = pl.BlockSpec((1, 1, K, BN), lambda b, i: (b, i, 0, 0))
    ef = jax.ShapeDtypeStruct((B * N, K), jnp.int32)
    eff = jax.ShapeDtypeStruct((B * N, K), jnp.float32)
    espec = pl.BlockSpec((BN, K), lambda b, i: (b * NB + i, 0))
    return pl.pallas_call(
        _tc_pre_body,
        grid=(B, NB),
        in_specs=[
            pl.BlockSpec((1, 1, CIN, BN), lambda b, i: (b, i, 0, 0)),
            bkn, bkn, bkn,
            pl.BlockSpec((1, 1, 3, K, BN), lambda b, i: (b, i, 0, 0, 0)),
            pl.BlockSpec((1, 1, BN, K), lambda b, i: (b, i, 0, 0)),
            pl.BlockSpec((COUT, CIN + 3), lambda b, i: (0, 0)),
        ],
        out_specs=[
            pl.BlockSpec((BN, COUT), lambda b, i: (b * NB + i, 0)),
            espec, espec, espec, espec, espec,
        ],
        out_shape=[
            jax.ShapeDtypeStruct((B * N, COUT), jnp.float32),
            ef, ef, eff, eff, eff,
        ],
    )(fea4, sx4, sy4, sz4, rel5, knn4, w1)


_BCAST_DN = lax.GatherDimensionNumbers(
    offset_dims=(), collapsed_slice_dims=(0,), start_index_map=(0,))


def _lane_bcast(vec, k):
    idx = jnp.full((16, 1), k, jnp.int32)
    return lax.gather(vec, idx, _BCAST_DN, slice_sizes=(1,),
                      mode=lax.GatherScatterMode.PROMISE_IN_BOUNDS)


def _sc_body(g_hbm, idx_hbm, cell_hbm, rx_hbm, ry_hbm, rz_hbm, wtab_hbm,
             w1xt_hbm, out_hbm,
             idx_v0, idx_v1, cell_v0, cell_v1, rx_v0, rx_v1, ry_v0, ry_v1,
             rz_v0, rz_v1, rows_v0, rows_v1, out_v0, out_v1, wtab_v, w1xt_v,
             si0, si1, sg0, sg1, so0, so1):
    idx_v = (idx_v0, idx_v1)
    cell_v = (cell_v0, cell_v1)
    rx_v = (rx_v0, rx_v1)
    ry_v = (ry_v0, ry_v1)
    rz_v = (rz_v0, rz_v1)
    rows_v = (rows_v0, rows_v1)
    out_v = (out_v0, out_v1)
    si = (si0, si1)
    sg = (sg0, sg1)
    so = (so0, so1)

    cid = lax.axis_index("c")
    sid = lax.axis_index("s")
    lid = cid * NS + sid
    units = jnp.where(lid < 17, 20, 19)
    bu = jnp.where(lid < 17, 20 * lid, 340 + 19 * (lid - 17))
    pb0 = 32 * bu
    nch = 2 * units
    pltpu.sync_copy(wtab_hbm, wtab_v)
    pltpu.sync_copy(w1xt_hbm, w1xt_v)
    w1r = [[w1xt_v[d, pl.ds(j * 16, 16)] for j in range(4)] for d in range(3)]

    def issue_in(c, b):
        pbase = pb0 + c * CP
        pltpu.async_copy(idx_hbm.at[pl.ds(pbase * K, CE)], idx_v[b], si[b])
        pltpu.async_copy(cell_hbm.at[pl.ds(pbase, CP)], cell_v[b], si[b])
        pltpu.async_copy(rx_hbm.at[pl.ds(pbase, CP)], rx_v[b], si[b])
        pltpu.async_copy(ry_hbm.at[pl.ds(pbase, CP)], ry_v[b], si[b])
        pltpu.async_copy(rz_hbm.at[pl.ds(pbase, CP)], rz_v[b], si[b])

    def wait_in(b):
        pltpu.make_async_copy(idx_hbm.at[pl.ds(0, CE)], idx_v[b], si[b]).wait()
        pltpu.make_async_copy(cell_hbm.at[pl.ds(0, CP)], cell_v[b], si[b]).wait()
        pltpu.make_async_copy(rx_hbm.at[pl.ds(0, CP)], rx_v[b], si[b]).wait()
        pltpu.make_async_copy(ry_hbm.at[pl.ds(0, CP)], ry_v[b], si[b]).wait()
        pltpu.make_async_copy(rz_hbm.at[pl.ds(0, CP)], rz_v[b], si[b]).wait()

    def issue_gather(b):
        for h in range(NG):
            hs = pl.ds(h * GE, GE)
            pltpu.async_copy(g_hbm.at[idx_v[b].at[hs]], rows_v[b].at[hs], sg[b])

    def wait_gather(b):
        for h in range(NG):
            hs = pl.ds(h * GE, GE)
            pltpu.make_async_copy(g_hbm.at[idx_v[b].at[hs]],
                                  rows_v[b].at[hs], sg[b]).wait()

    def issue_out(c, b):
        pbase = pb0 + c * CP
        pltpu.async_copy(out_v[b], out_hbm.at[pl.ds(pbase, CP)], so[b])

    def wait_out(b):
        pltpu.make_async_copy(out_v[b], out_hbm.at[pl.ds(0, CP)], so[b]).wait()

    def compute(b):
        def point_body(p, pcarry):
            base = p * K
            cv = cell_v[b][p]
            rxv = rx_v[b][p]
            ryv = ry_v[b][p]
            rzv = rz_v[b][p]
            accs = [jnp.zeros((16,), jnp.float32) for _ in range(4)]
            for k in range(K):
                cl = cv[k]
                rxb = _lane_bcast(rxv, k)
                ryb = _lane_bcast(ryv, k)
                rzb = _lane_bcast(rzv, k)
                row = base + k
                for j in range(4):
                    jds = pl.ds(j * 16, 16)
                    w = wtab_v[cl, jds]
                    u = (rows_v[b][row, jds] + rxb * w1r[0][j] +
                         ryb * w1r[1][j] + rzb * w1r[2][j])
                    accs[j] = accs[j] + w * u
            for j in range(4):
                out_v[b][p, pl.ds(j * 16, 16)] = accs[j]
            return pcarry

        lax.fori_loop(0, CP, point_body, 0)

    issue_in(0, 0)
    wait_in(0)
    issue_gather(0)
    issue_in(1, 1)

    def body2(c2, carry):
        for b in range(2):
            c = c2 * 2 + b
            nb = 1 - b
            wait_gather(b)

            @pl.when(c + 1 < nch)
            def _():
                wait_in(nb)
                issue_gather(nb)

            @pl.when(c >= 2)
            def _():
                wait_out(b)

            compute(b)
            issue_out(c, b)

            @pl.when(c + 2 < nch)
            def _():
                issue_in(c + 2, b)
        return carry

    lax.fori_loop(0, nch // 2, body2, 0)
    wait_out(0)
    wait_out(1)


def _sc_gather_combine(g, idx_flat, cell_pad, rx, ry, rz, wtab, w1xt):
    mesh = plsc.VectorSubcoreMesh(core_axis_name="c", subcore_axis_name="s")
    cpk = pltpu.VMEM((CP, K), jnp.int32)
    cpf = pltpu.VMEM((CP, K), jnp.float32)
    f = functools.partial(
        pl.kernel,
        mesh=mesh,
        compiler_params=pltpu.CompilerParams(use_tc_tiling_on_sc=False),
        out_type=jax.ShapeDtypeStruct((B * N, COUT), jnp.float32),
        scratch_types=[
            pltpu.VMEM((CE,), jnp.int32), pltpu.VMEM((CE,), jnp.int32),
            cpk, cpk,
            cpf, cpf, cpf, cpf, cpf, cpf,
            pltpu.VMEM((CE, COUT), jnp.float32),
            pltpu.VMEM((CE, COUT), jnp.float32),
            pltpu.VMEM((CP, COUT), jnp.float32),
            pltpu.VMEM((CP, COUT), jnp.float32),
            pltpu.VMEM((27, COUT), jnp.float32),
            pltpu.VMEM((3, COUT), jnp.float32),
            pltpu.SemaphoreType.DMA, pltpu.SemaphoreType.DMA,
            pltpu.SemaphoreType.DMA, pltpu.SemaphoreType.DMA,
            pltpu.SemaphoreType.DMA, pltpu.SemaphoreType.DMA,
        ],
    )(_sc_body)
    return f(g, idx_flat, cell_pad, rx, ry, rz, wtab, w1xt)


def _tc_post_body(sc_ref, out_ref):
    out_ref[0] = sc_ref[...].T


def _tc_post(out_sc):
    return pl.pallas_call(
        _tc_post_body,
        grid=(B,),
        in_specs=[pl.BlockSpec((N, COUT), lambda b: (b, 0))],
        out_specs=pl.BlockSpec((1, COUT, N), lambda b: (b, 0, 0)),
        out_shape=jax.ShapeDtypeStruct((B, COUT, N), jnp.float32),
    )(out_sc)


def kernel(rel_xyz, sample_xyz, fea, knn_idx, conv_dw, W1):
    wtab = conv_dw.reshape(COUT, 27).T
    w1xt = W1[:, CIN:].T
    sq = jnp.squeeze(sample_xyz, 3)
    sx4 = jnp.transpose(sq[..., 0].reshape(B, K, NB, BN), (0, 2, 1, 3))
    sy4 = jnp.transpose(sq[..., 1].reshape(B, K, NB, BN), (0, 2, 1, 3))
    sz4 = jnp.transpose(sq[..., 2].reshape(B, K, NB, BN), (0, 2, 1, 3))
    rel5 = jnp.transpose(rel_xyz.reshape(B, 3, K, NB, BN), (0, 3, 1, 2, 4))
    fea4 = jnp.transpose(fea.reshape(B, CIN, NB, BN), (0, 2, 1, 3))
    knn4 = knn_idx.reshape(B, NB, BN, K)

    g, idx_e, cell_e, rx_e, ry_e, rz_e = _tc_pre(
        fea4, sx4, sy4, sz4, rel5, knn4, W1)

    idx_flat = idx_e.reshape(B * N * K)
    out_sc = _sc_gather_combine(
        g, idx_flat, cell_e, rx_e, ry_e, rz_e, wtab, w1xt)
    return _tc_post(out_sc)
